# Initial kernel scaffold; baseline (speedup 1.0000x reference)
import jax
import jax.numpy as jnp
from jax.experimental import pallas as pl


def kernel(source, target):
    raise NotImplementedError("write your pallas kernel here")


# speedup vs baseline: 1.9825x; 1.9825x over previous
import jax
import jax.numpy as jnp
from jax.experimental import pallas as pl

_N = 2048
_QB = 256
_ITERS = 5


def _polar3(M):
    A = [[M[0][i] * M[0][j] + M[1][i] * M[1][j] + M[2][i] * M[2][j]
          for j in range(3)] for i in range(3)]
    V = [[1.0 if i == j else 0.0 for j in range(3)] for i in range(3)]
    for _ in range(4):
        for (p, q) in ((0, 1), (0, 2), (1, 2)):
            apq = A[p][q]
            app = A[p][p]
            aqq = A[q][q]
            denom = 2.0 * apq
            tau = (aqq - app) / jnp.where(denom == 0.0, 1.0, denom)
            sgn = jnp.where(tau >= 0.0, 1.0, -1.0)
            t = sgn / (jnp.abs(tau) + jnp.sqrt(1.0 + tau * tau))
            t = jnp.where(apq == 0.0, 0.0, t)
            c = 1.0 / jnp.sqrt(1.0 + t * t)
            s = t * c
            r = 3 - p - q
            arp = A[r][p]
            arq = A[r][q]
            A[p][p] = app - t * apq
            A[q][q] = aqq + t * apq
            A[p][q] = A[q][p] = jnp.zeros_like(apq)
            A[r][p] = A[p][r] = c * arp - s * arq
            A[r][q] = A[q][r] = s * arp + c * arq
            for i in range(3):
                vip = V[i][p]
                viq = V[i][q]
                V[i][p] = c * vip - s * viq
                V[i][q] = s * vip + c * viq
    inv_s = [1.0 / jnp.sqrt(jnp.maximum(A[k][k], 1e-30)) for k in range(3)]
    W = [[V[i][0] * inv_s[0] * V[j][0]
          + V[i][1] * inv_s[1] * V[j][1]
          + V[i][2] * inv_s[2] * V[j][2] for j in range(3)] for i in range(3)]
    R = [[M[i][0] * W[0][j] + M[i][1] * W[1][j] + M[i][2] * W[2][j]
          for j in range(3)] for i in range(3)]
    E = [[R[0][i] * R[0][j] + R[1][i] * R[1][j] + R[2][i] * R[2][j]
          for j in range(3)] for i in range(3)]
    B = [[(1.5 if i == j else 0.0) - 0.5 * E[i][j] for j in range(3)]
         for i in range(3)]
    R = [[R[i][0] * B[0][j] + R[i][1] * B[1][j] + R[i][2] * B[2][j]
          for j in range(3)] for i in range(3)]
    det = (R[0][0] * (R[1][1] * R[2][2] - R[1][2] * R[2][1])
           - R[0][1] * (R[1][0] * R[2][2] - R[1][2] * R[2][0])
           + R[0][2] * (R[1][0] * R[2][1] - R[1][1] * R[2][0]))
    flip = jnp.where(jnp.abs(det + 1.0) < 1e-6, -1.0, 1.0)
    return [[R[i][j] * flip for j in range(3)] for i in range(3)]


def _rigid(S2, ct, cs, n):
    M = [[S2[a][b] - n * ct[a] * cs[b] for b in range(3)] for a in range(3)]
    R = _polar3(M)
    tv = [ct[a] - (R[a][0] * cs[0] + R[a][1] * cs[1] + R[a][2] * cs[2])
          for a in range(3)]
    return R, tv


def _icp_body(src_ref, tgt_t_ref, out_ref):
    src = src_ref[0]
    tgt_t = tgt_t_ref[0]
    g = [tgt_t[d:d + 1, :] for d in range(3)]
    s_cols = [src[:, d:d + 1] for d in range(3)]
    cols = list(s_cols)
    n = float(_N)
    cs_src = [jnp.sum(c, keepdims=True) / n for c in s_cols]

    for _ in range(_ITERS):
        S1 = [jnp.zeros((1, 1), jnp.float32) for _ in range(3)]
        S2 = [[jnp.zeros((1, 1), jnp.float32) for _ in range(3)]
              for _ in range(3)]
        for blk in range(_N // _QB):
            sl = slice(blk * _QB, (blk + 1) * _QB)
            q = [cols[d][sl] for d in range(3)]
            dist = ((q[0] - g[0]) ** 2 + (q[1] - g[1]) ** 2
                    + (q[2] - g[2]) ** 2)
            m = jnp.min(dist, axis=1, keepdims=True)
            mf = (dist <= m).astype(jnp.float32)
            inv = 1.0 / jnp.sum(mf, axis=1, keepdims=True)
            b = [jnp.sum(mf * g[d], axis=1, keepdims=True) * inv
                 for d in range(3)]
            for a in range(3):
                S1[a] = S1[a] + jnp.sum(b[a], keepdims=True)
                for cdim in range(3):
                    S2[a][cdim] = (S2[a][cdim]
                                   + jnp.sum(b[a] * q[cdim], keepdims=True))
        ct = [S1[a] / n for a in range(3)]
        cs = [jnp.sum(cols[d], keepdims=True) / n for d in range(3)]
        R, tv = _rigid(S2, ct, cs, n)
        cols = [R[a][0] * cols[0] + R[a][1] * cols[1] + R[a][2] * cols[2]
                + tv[a] for a in range(3)]

    ct = [jnp.sum(cols[d], keepdims=True) / n for d in range(3)]
    S2 = [[jnp.sum(cols[a] * s_cols[b], keepdims=True) for b in range(3)]
          for a in range(3)]
    R, tv = _rigid(S2, ct, cs_src, n)
    rows = [jnp.concatenate([R[a][0], R[a][1], R[a][2], tv[a]], axis=1)
            for a in range(3)]
    out_ref[0] = jnp.concatenate(rows, axis=0)


def kernel(source, target):
    batch = source.shape[0]
    tgt_t = jnp.swapaxes(target, -1, -2)
    return pl.pallas_call(
        _icp_body,
        grid=(batch,),
        in_specs=[
            pl.BlockSpec((1, _N, 3), lambda b: (b, 0, 0)),
            pl.BlockSpec((1, 3, _N), lambda b: (b, 0, 0)),
        ],
        out_specs=pl.BlockSpec((1, 3, 4), lambda b: (b, 0, 0)),
        out_shape=jax.ShapeDtypeStruct((batch, 3, 4), jnp.float32),
    )(source, tgt_t)


# speedup vs baseline: 2.0941x; 1.0563x over previous
import jax
import jax.numpy as jnp
from jax.experimental import pallas as pl

_N = 2048
_QB = 512
_ITERS = 5


def _polar3(M):
    A = [[M[0][i] * M[0][j] + M[1][i] * M[1][j] + M[2][i] * M[2][j]
          for j in range(3)] for i in range(3)]
    V = [[1.0 if i == j else 0.0 for j in range(3)] for i in range(3)]
    for _ in range(4):
        for (p, q) in ((0, 1), (0, 2), (1, 2)):
            apq = A[p][q]
            app = A[p][p]
            aqq = A[q][q]
            denom = 2.0 * apq
            tau = (aqq - app) / jnp.where(denom == 0.0, 1.0, denom)
            sgn = jnp.where(tau >= 0.0, 1.0, -1.0)
            t = sgn / (jnp.abs(tau) + jnp.sqrt(1.0 + tau * tau))
            t = jnp.where(apq == 0.0, 0.0, t)
            c = 1.0 / jnp.sqrt(1.0 + t * t)
            s = t * c
            r = 3 - p - q
            arp = A[r][p]
            arq = A[r][q]
            A[p][p] = app - t * apq
            A[q][q] = aqq + t * apq
            A[p][q] = A[q][p] = jnp.zeros_like(apq)
            A[r][p] = A[p][r] = c * arp - s * arq
            A[r][q] = A[q][r] = s * arp + c * arq
            for i in range(3):
                vip = V[i][p]
                viq = V[i][q]
                V[i][p] = c * vip - s * viq
                V[i][q] = s * vip + c * viq
    inv_s = [1.0 / jnp.sqrt(jnp.maximum(A[k][k], 1e-30)) for k in range(3)]
    W = [[V[i][0] * inv_s[0] * V[j][0]
          + V[i][1] * inv_s[1] * V[j][1]
          + V[i][2] * inv_s[2] * V[j][2] for j in range(3)] for i in range(3)]
    R = [[M[i][0] * W[0][j] + M[i][1] * W[1][j] + M[i][2] * W[2][j]
          for j in range(3)] for i in range(3)]
    E = [[R[0][i] * R[0][j] + R[1][i] * R[1][j] + R[2][i] * R[2][j]
          for j in range(3)] for i in range(3)]
    B = [[(1.5 if i == j else 0.0) - 0.5 * E[i][j] for j in range(3)]
         for i in range(3)]
    R = [[R[i][0] * B[0][j] + R[i][1] * B[1][j] + R[i][2] * B[2][j]
          for j in range(3)] for i in range(3)]
    det = (R[0][0] * (R[1][1] * R[2][2] - R[1][2] * R[2][1])
           - R[0][1] * (R[1][0] * R[2][2] - R[1][2] * R[2][0])
           + R[0][2] * (R[1][0] * R[2][1] - R[1][1] * R[2][0]))
    flip = jnp.where(jnp.abs(det + 1.0) < 1e-6, -1.0, 1.0)
    return [[R[i][j] * flip for j in range(3)] for i in range(3)]


def _rigid(S2, ct, cs, n):
    M = [[S2[a][b] - n * ct[a] * cs[b] for b in range(3)] for a in range(3)]
    R = _polar3(M)
    tv = [ct[a] - (R[a][0] * cs[0] + R[a][1] * cs[1] + R[a][2] * cs[2])
          for a in range(3)]
    return R, tv


def _icp_body(src_ref, tgt_ref, tgt_t_ref, out_ref):
    src = src_ref[0]
    tgt = tgt_ref[0]
    tgt_t = tgt_t_ref[0]
    tn = (tgt_t[0:1, :] ** 2 + tgt_t[1:2, :] ** 2
          + tgt_t[2:3, :] ** 2)
    n = float(_N)
    cs_src = [jnp.sum(src[:, d:d + 1], keepdims=True) / n for d in range(3)]
    T = src

    for _ in range(_ITERS):
        S1 = [jnp.zeros((1, 1), jnp.float32) for _ in range(3)]
        S2 = [[jnp.zeros((1, 1), jnp.float32) for _ in range(3)]
              for _ in range(3)]
        for blk in range(_N // _QB):
            q = T[blk * _QB:(blk + 1) * _QB]
            G = jnp.dot(q, tgt_t,
                        preferred_element_type=jnp.float32)
            dist = tn - 2.0 * G
            m = jnp.min(dist, axis=1, keepdims=True)
            mf = (dist <= m).astype(jnp.float32)
            inv = 1.0 / jnp.sum(mf, axis=1, keepdims=True)
            bsum = jnp.dot(mf, tgt,
                           preferred_element_type=jnp.float32)
            bpt = bsum * inv
            for a in range(3):
                S1[a] = S1[a] + jnp.sum(bpt[:, a:a + 1], keepdims=True)
                for c in range(3):
                    S2[a][c] = S2[a][c] + jnp.sum(
                        bpt[:, a:a + 1] * q[:, c:c + 1], keepdims=True)
        ct = [S1[a] / n for a in range(3)]
        cs = [jnp.sum(T[:, d:d + 1], keepdims=True) / n for d in range(3)]
        R, tv = _rigid(S2, ct, cs, n)
        c0 = T[:, 0:1]
        c1 = T[:, 1:2]
        c2 = T[:, 2:3]
        T = jnp.concatenate(
            [R[a][0] * c0 + R[a][1] * c1 + R[a][2] * c2 + tv[a]
             for a in range(3)], axis=1)

    ct = [jnp.sum(T[:, d:d + 1], keepdims=True) / n for d in range(3)]
    S2 = [[jnp.sum(T[:, a:a + 1] * src[:, c:c + 1], keepdims=True)
           for c in range(3)] for a in range(3)]
    R, tv = _rigid(S2, ct, cs_src, n)
    rows = [jnp.concatenate([R[a][0], R[a][1], R[a][2], tv[a]], axis=1)
            for a in range(3)]
    out_ref[0] = jnp.concatenate(rows, axis=0)


def kernel(source, target):
    batch = source.shape[0]
    tgt_t = jnp.swapaxes(target, -1, -2)
    return pl.pallas_call(
        _icp_body,
        grid=(batch,),
        in_specs=[
            pl.BlockSpec((1, _N, 3), lambda b: (b, 0, 0)),
            pl.BlockSpec((1, _N, 3), lambda b: (b, 0, 0)),
            pl.BlockSpec((1, 3, _N), lambda b: (b, 0, 0)),
        ],
        out_specs=pl.BlockSpec((1, 3, 4), lambda b: (b, 0, 0)),
        out_shape=jax.ShapeDtypeStruct((batch, 3, 4), jnp.float32),
    )(source, target, tgt_t)
